# Initial kernel scaffold; baseline (speedup 1.0000x reference)
import jax
import jax.numpy as jnp
from jax.experimental import pallas as pl


def kernel(node_feat, edge_index, edge_weight, h, Wz, bz, Wr, br, Wh, bh, Wlz, blz, Wlr, blr, Wlh, blh, Wout, bout):
    raise NotImplementedError("write your pallas kernel here")


# speedup vs baseline: 16.7426x; 16.7426x over previous
import dataclasses
import functools

import jax
import jax.numpy as jnp
from jax import lax
from jax.experimental import pallas as pl
from jax.experimental.pallas import tpu as pltpu
from jax.experimental.pallas import tpu_sc as plsc

NC = 2
NS = 16
NW = NC * NS
LANE = 16
CHUNK = 128
IB = 8

_mesh = plsc.VectorSubcoreMesh(core_axis_name="c", subcore_axis_name="s")

_sc_params = pltpu.CompilerParams()
if "needs_layout_passes" in pltpu.CompilerParams.__dataclass_fields__:
    _sc_params = dataclasses.replace(_sc_params, needs_layout_passes=False)


def _make_deg_kernel(n_nodes, rows_total):
    rt = rows_total // NW
    nz = (n_nodes // NS) // 8 * 8
    rem0 = nz * NS
    rem = n_nodes - rem0

    @functools.partial(
        pl.kernel,
        mesh=_mesh,
        out_type=jax.ShapeDtypeStruct((NC * n_nodes, LANE), jnp.float32),
        compiler_params=_sc_params,
        scratch_types=[
            pltpu.VMEM((IB, CHUNK), jnp.int32),
            pltpu.VMEM((IB, CHUNK), jnp.float32),
            pltpu.VMEM((CHUNK, LANE), jnp.float32),
            pltpu.VMEM_SHARED((n_nodes, LANE), jnp.float32),
        ],
    )
    def deg_kernel(dst_hbm, w_hbm, zeros_hbm, out_hbm, dst_v, w_v, stage_v,
                   deg_sp):
        cid = lax.axis_index("c")
        sid = lax.axis_index("s")
        pltpu.sync_copy(zeros_hbm.at[pl.ds(sid * nz, nz)],
                        deg_sp.at[pl.ds(sid * nz, nz)])
        if rem:
            @pl.when(sid == 0)
            def _():
                pltpu.sync_copy(zeros_hbm.at[pl.ds(rem0, rem)],
                                deg_sp.at[pl.ds(rem0, rem)])
        plsc.subcore_barrier()
        wid = cid * NS + sid
        base_row = wid * rt

        @pl.loop(0, rt // IB)
        def _(bi):
            r0 = base_row + bi * IB
            pltpu.sync_copy(dst_hbm.at[pl.ds(r0, IB)], dst_v)
            pltpu.sync_copy(w_hbm.at[pl.ds(r0, IB)], w_v)
            for j in range(IB):
                for g in range(CHUNK // LANE):
                    wv = w_v[j, pl.ds(g * LANE, LANE)]
                    for t in range(LANE):
                        stage_v[g * LANE + t, pl.ds(0, LANE)] = jnp.full(
                            (LANE,), wv[t], dtype=jnp.float32)
                pltpu.sync_copy(stage_v, deg_sp.at[dst_v.at[j]], add=True)

        plsc.subcore_barrier()
        pltpu.sync_copy(deg_sp.at[pl.ds(sid * nz, nz)],
                        out_hbm.at[pl.ds(cid * n_nodes + sid * nz, nz)])
        if rem:
            @pl.when(sid == 0)
            def _():
                pltpu.sync_copy(deg_sp.at[pl.ds(rem0, rem)],
                                out_hbm.at[pl.ds(cid * n_nodes + rem0, rem)])

    return deg_kernel


def _make_agg_kernel(n_nodes, d, rows_total):
    rt = rows_total // NW
    nz = (n_nodes // NS) // 8 * 8
    rem0 = nz * NS
    rem = n_nodes - rem0

    @functools.partial(
        pl.kernel,
        mesh=_mesh,
        out_type=jax.ShapeDtypeStruct((NC * n_nodes, d), jnp.float32),
        compiler_params=_sc_params,
        scratch_types=[
            pltpu.VMEM((IB, CHUNK), jnp.int32),
            pltpu.VMEM((IB, CHUNK), jnp.int32),
            pltpu.VMEM((IB, CHUNK), jnp.float32),
            pltpu.VMEM((CHUNK, d), jnp.float32),
            pltpu.VMEM_SHARED((n_nodes, d), jnp.float32),
        ],
    )
    def agg_kernel(src_hbm, dst_hbm, w_hbm, xs_hbm, zeros_hbm, out_hbm,
                   src_v, dst_v, w_v, rows_v, acc_sp):
        cid = lax.axis_index("c")
        sid = lax.axis_index("s")
        pltpu.sync_copy(zeros_hbm.at[pl.ds(sid * nz, nz)],
                        acc_sp.at[pl.ds(sid * nz, nz)])
        if rem:
            @pl.when(sid == 0)
            def _():
                pltpu.sync_copy(zeros_hbm.at[pl.ds(rem0, rem)],
                                acc_sp.at[pl.ds(rem0, rem)])
        plsc.subcore_barrier()
        wid = cid * NS + sid
        base_row = wid * rt

        @pl.loop(0, rt // IB)
        def _(bi):
            r0 = base_row + bi * IB
            pltpu.sync_copy(src_hbm.at[pl.ds(r0, IB)], src_v)
            pltpu.sync_copy(dst_hbm.at[pl.ds(r0, IB)], dst_v)
            pltpu.sync_copy(w_hbm.at[pl.ds(r0, IB)], w_v)
            for j in range(IB):
                pltpu.sync_copy(xs_hbm.at[src_v.at[j]], rows_v)

                @pl.loop(0, CHUNK // LANE)
                def _(g):
                    wv = w_v[j, pl.ds(g * LANE, LANE)]
                    for t in range(LANE):
                        wsplat = jnp.full((LANE,), wv[t], dtype=jnp.float32)
                        e = g * LANE + t
                        for k in range(d // LANE):
                            sl = pl.ds(k * LANE, LANE)
                            rows_v[e, sl] = rows_v[e, sl] * wsplat

                pltpu.sync_copy(rows_v, acc_sp.at[dst_v.at[j]], add=True)

        plsc.subcore_barrier()
        pltpu.sync_copy(acc_sp.at[pl.ds(sid * nz, nz)],
                        out_hbm.at[pl.ds(cid * n_nodes + sid * nz, nz)])
        if rem:
            @pl.when(sid == 0)
            def _():
                pltpu.sync_copy(acc_sp.at[pl.ds(rem0, rem)],
                                out_hbm.at[pl.ds(cid * n_nodes + rem0, rem)])

    return agg_kernel


def _prep_body(degt_ref, x_ref, dinv_ref, xs_ref):
    deg = degt_ref[0, :, 0:1] + degt_ref[1, :, 0:1] + 1.0
    dinv = lax.rsqrt(deg)
    dinv_ref[...] = dinv
    xs_ref[...] = x_ref[...] * dinv


def _gates_body(acc_ref, xs_ref, dinv_ref, h_ref,
                Wz_ref, Wr_ref, Wh_ref, Wlz_ref, Wlr_ref, Wlh_ref,
                bz_ref, br_ref, bh_ref, blz_ref, blr_ref, blh_ref,
                Wout_ref, bout_ref, z_ref, H_ref):
    d = xs_ref.shape[1]
    hi = pl.ds(0, d)
    lo = pl.ds(d, d)

    def mm(a, b):
        return lax.dot_general(a, b, (((1,), (0,)), ((), ())),
                               precision=lax.Precision.HIGHEST,
                               preferred_element_type=jnp.float32)

    agg = (acc_ref[0] + acc_ref[1] + xs_ref[...]) * dinv_ref[...]
    h = h_ref[...]

    Az = mm(Wz_ref[...], Wlz_ref[hi, :])
    cz = mm(bz_ref[...], Wlz_ref[hi, :]) + blz_ref[...]
    Z = jax.nn.sigmoid(mm(agg, Az) + mm(h, Wlz_ref[lo, :]) + cz)

    Ar = mm(Wr_ref[...], Wlr_ref[hi, :])
    cr = mm(br_ref[...], Wlr_ref[hi, :]) + blr_ref[...]
    R = jax.nn.sigmoid(mm(agg, Ar) + mm(h, Wlr_ref[lo, :]) + cr)

    Ah = mm(Wh_ref[...], Wlh_ref[hi, :])
    ch = mm(bh_ref[...], Wlh_ref[hi, :]) + blh_ref[...]
    C = jnp.tanh(mm(agg, Ah) + mm(h * R, Wlh_ref[lo, :]) + ch)

    H = Z * h + (1.0 - Z) * C
    H_ref[...] = H
    z_ref[...] = mm(jnp.maximum(H, 0.0), Wout_ref[...]) + bout_ref[...]


def kernel(node_feat, edge_index, edge_weight, h,
           Wz, bz, Wr, br, Wh, bh,
           Wlz, blz, Wlr, blr, Wlh, blh, Wout, bout):
    B, Nn, Din = node_feat.shape
    n = B * Nn
    d = h.shape[1]
    x = node_feat.reshape(n, Din)
    src = edge_index[0]
    dst = edge_index[1]
    E = src.shape[0]

    group = NW * CHUNK * IB
    epad = ((E + group - 1) // group) * group
    pad = epad - E
    srcp = jnp.concatenate([src, jnp.zeros((pad,), jnp.int32)]).reshape(-1, CHUNK)
    dstp = jnp.concatenate([dst, jnp.zeros((pad,), jnp.int32)]).reshape(-1, CHUNK)
    wp = jnp.concatenate(
        [edge_weight, jnp.zeros((pad,), jnp.float32)]).reshape(-1, CHUNK)
    rows_total = epad // CHUNK

    zeros_nl = jnp.zeros((n, LANE), jnp.float32)
    zeros_nd = jnp.zeros((n, d), jnp.float32)

    degp = _make_deg_kernel(n, rows_total)(dstp, wp, zeros_nl)
    degt = degp.reshape(NC, n, LANE)

    R = 2000
    grid = (n // R,)
    dinv, xs = pl.pallas_call(
        _prep_body,
        grid=grid,
        in_specs=[
            pl.BlockSpec((NC, R, LANE), lambda i: (0, i, 0)),
            pl.BlockSpec((R, Din), lambda i: (i, 0)),
        ],
        out_specs=[
            pl.BlockSpec((R, 1), lambda i: (i, 0)),
            pl.BlockSpec((R, Din), lambda i: (i, 0)),
        ],
        out_shape=[
            jax.ShapeDtypeStruct((n, 1), jnp.float32),
            jax.ShapeDtypeStruct((n, Din), jnp.float32),
        ],
    )(degt, x)

    accp = _make_agg_kernel(n, d, rows_total)(srcp, dstp, wp, xs, zeros_nd)
    accp = accp.reshape(NC, n, d)

    full = lambda *shape: pl.BlockSpec(shape, lambda i: tuple(0 for _ in shape))
    row_blk = lambda c: pl.BlockSpec((R, c), lambda i: (i, 0))
    z, H = pl.pallas_call(
        _gates_body,
        grid=grid,
        in_specs=[
            pl.BlockSpec((NC, R, d), lambda i: (0, i, 0)),
            row_blk(d), row_blk(1), row_blk(d),
            full(d, d), full(d, d), full(d, d),
            full(2 * d, d), full(2 * d, d), full(2 * d, d),
            full(1, d), full(1, d), full(1, d),
            full(1, d), full(1, d), full(1, d),
            full(d, 1), full(1, 1),
        ],
        out_specs=[row_blk(1), row_blk(d)],
        out_shape=[
            jax.ShapeDtypeStruct((n, 1), jnp.float32),
            jax.ShapeDtypeStruct((n, d), jnp.float32),
        ],
    )(accp, xs, dinv, h, Wz, Wr, Wh, Wlz, Wlr, Wlh,
      bz.reshape(1, d), br.reshape(1, d), bh.reshape(1, d),
      blz.reshape(1, d), blr.reshape(1, d), blh.reshape(1, d),
      Wout, bout.reshape(1, 1))

    return z.reshape(B, Nn, 1), H


# speedup vs baseline: 18.5904x; 1.1104x over previous
import dataclasses
import functools

import jax
import jax.numpy as jnp
from jax import lax
from jax.experimental import pallas as pl
from jax.experimental.pallas import tpu as pltpu
from jax.experimental.pallas import tpu_sc as plsc

NC = 2
NS = 16
NW = NC * NS
LANE = 16
CHUNK = 32
IB = 8
PHR = 40

_mesh = plsc.VectorSubcoreMesh(core_axis_name="c", subcore_axis_name="s")

_sc_params = pltpu.CompilerParams()
if "needs_layout_passes" in pltpu.CompilerParams.__dataclass_fields__:
    _sc_params = dataclasses.replace(_sc_params, needs_layout_passes=False)


def _make_deg_kernel(n_nodes, rows_total):
    rt = rows_total // NW
    nz = (n_nodes // NS) // 8 * 8
    rem0 = nz * NS
    rem = n_nodes - rem0

    @functools.partial(
        pl.kernel,
        mesh=_mesh,
        out_type=jax.ShapeDtypeStruct((NC * n_nodes, LANE), jnp.float32),
        compiler_params=_sc_params,
        scratch_types=[
            pltpu.VMEM((IB, CHUNK), jnp.int32),
            pltpu.VMEM((IB, CHUNK), jnp.float32),
            pltpu.VMEM((CHUNK, LANE), jnp.float32),
            pltpu.VMEM_SHARED((n_nodes, LANE), jnp.float32),
        ],
    )
    def deg_kernel(dst_hbm, w_hbm, zeros_hbm, out_hbm, dst_v, w_v, stage_v,
                   deg_sp):
        cid = lax.axis_index("c")
        sid = lax.axis_index("s")
        pltpu.sync_copy(zeros_hbm.at[pl.ds(sid * nz, nz)],
                        deg_sp.at[pl.ds(sid * nz, nz)])
        if rem:
            @pl.when(sid == 0)
            def _():
                pltpu.sync_copy(zeros_hbm.at[pl.ds(rem0, rem)],
                                deg_sp.at[pl.ds(rem0, rem)])
        plsc.subcore_barrier()
        wid = cid * NS + sid
        base_row = wid * rt

        @pl.loop(0, rt // IB)
        def _(bi):
            r0 = base_row + bi * IB
            pltpu.sync_copy(dst_hbm.at[pl.ds(r0, IB)], dst_v)
            pltpu.sync_copy(w_hbm.at[pl.ds(r0, IB)], w_v)
            for j in range(IB):
                for g in range(CHUNK // LANE):
                    wv = w_v[j, pl.ds(g * LANE, LANE)]
                    for t in range(LANE):
                        stage_v[g * LANE + t, pl.ds(0, LANE)] = jnp.full(
                            (LANE,), wv[t], dtype=jnp.float32)
                pltpu.sync_copy(stage_v, deg_sp.at[dst_v.at[j]], add=True)

        plsc.subcore_barrier()
        pltpu.sync_copy(deg_sp.at[pl.ds(sid * nz, nz)],
                        out_hbm.at[pl.ds(cid * n_nodes + sid * nz, nz)])
        if rem:
            @pl.when(sid == 0)
            def _():
                pltpu.sync_copy(deg_sp.at[pl.ds(rem0, rem)],
                                out_hbm.at[pl.ds(cid * n_nodes + rem0, rem)])

    return deg_kernel


def _make_agg_kernel(n_nodes, d, rows_total):
    rt = rows_total // NW
    nph = rt // PHR
    nz = (n_nodes // NS) // 8 * 8
    rem0 = nz * NS
    rem = n_nodes - rem0

    @functools.partial(
        pl.kernel,
        mesh=_mesh,
        out_type=jax.ShapeDtypeStruct((NC * n_nodes, d), jnp.float32),
        compiler_params=_sc_params,
        scratch_types=[
            pltpu.VMEM((PHR, CHUNK), jnp.int32),
            pltpu.VMEM((PHR, CHUNK), jnp.int32),
            pltpu.VMEM((PHR, CHUNK), jnp.float32),
            pltpu.VMEM((CHUNK, d), jnp.float32),
            pltpu.VMEM((CHUNK, d), jnp.float32),
            pltpu.VMEM((CHUNK, d), jnp.float32),
            pltpu.VMEM((CHUNK, d), jnp.float32),
            pltpu.VMEM_SHARED((n_nodes, d), jnp.float32),
            pltpu.SemaphoreType.DMA,
            pltpu.SemaphoreType.DMA,
            pltpu.SemaphoreType.DMA,
            pltpu.SemaphoreType.DMA,
        ],
    )
    def agg_kernel(src_hbm, dst_hbm, w_hbm, xs_hbm, zeros_hbm, out_hbm,
                   src_v, dst_v, w_v, g0, g1, s0, s1, acc_sp,
                   gs0, gs1, ss0, ss1):
        cid = lax.axis_index("c")
        sid = lax.axis_index("s")
        wid = cid * NS + sid
        base_row = wid * rt
        pltpu.sync_copy(zeros_hbm.at[pl.ds(sid * nz, nz)],
                        acc_sp.at[pl.ds(sid * nz, nz)])
        if rem:
            @pl.when(sid == 0)
            def _():
                pltpu.sync_copy(zeros_hbm.at[pl.ds(rem0, rem)],
                                acc_sp.at[pl.ds(rem0, rem)])
        plsc.subcore_barrier()

        bufs = ((g0, s0, gs0, ss0), (g1, s1, gs1, ss1))

        for ph in range(nph):
            pb = base_row + ph * PHR
            if ph > 0:
                pltpu.make_async_copy(s0, acc_sp.at[dst_v.at[PHR - 2]],
                                      ss0).wait()
                pltpu.make_async_copy(s1, acc_sp.at[dst_v.at[PHR - 1]],
                                      ss1).wait()
            pltpu.sync_copy(src_hbm.at[pl.ds(pb, PHR)], src_v)
            pltpu.sync_copy(dst_hbm.at[pl.ds(pb, PHR)], dst_v)
            pltpu.sync_copy(w_hbm.at[pl.ds(pb, PHR)], w_v)
            pltpu.async_copy(xs_hbm.at[src_v.at[0]], g0, gs0)
            pltpu.async_copy(xs_hbm.at[src_v.at[1]], g1, gs1)

            @pl.loop(0, PHR // 2)
            def _(i):
                for p in range(2):
                    g, s, gsem, ssem = bufs[p]
                    r = 2 * i + p
                    pltpu.make_async_copy(xs_hbm.at[src_v.at[r]], g,
                                          gsem).wait()

                    @pl.when(i > 0)
                    def _():
                        pltpu.make_async_copy(s, acc_sp.at[dst_v.at[r]],
                                              ssem).wait()

                    @pl.loop(0, CHUNK // LANE)
                    def _(gi):
                        wv = w_v[r, pl.ds(gi * LANE, LANE)]
                        for t in range(LANE):
                            wsplat = jnp.full((LANE,), wv[t],
                                              dtype=jnp.float32)
                            e = gi * LANE + t
                            for k in range(d // LANE):
                                sl = pl.ds(k * LANE, LANE)
                                s[e, sl] = g[e, sl] * wsplat

                    pltpu.async_copy(s, acc_sp.at[dst_v.at[r]], ssem,
                                     add=True)

                    @pl.when(i < PHR // 2 - 1)
                    def _():
                        pltpu.async_copy(xs_hbm.at[src_v.at[r + 2]], g, gsem)

        pltpu.make_async_copy(s0, acc_sp.at[dst_v.at[PHR - 2]], ss0).wait()
        pltpu.make_async_copy(s1, acc_sp.at[dst_v.at[PHR - 1]], ss1).wait()
        plsc.subcore_barrier()
        pltpu.sync_copy(acc_sp.at[pl.ds(sid * nz, nz)],
                        out_hbm.at[pl.ds(cid * n_nodes + sid * nz, nz)])
        if rem:
            @pl.when(sid == 0)
            def _():
                pltpu.sync_copy(acc_sp.at[pl.ds(rem0, rem)],
                                out_hbm.at[pl.ds(cid * n_nodes + rem0, rem)])

    return agg_kernel


def _prep_body(degt_ref, x_ref, dinv_ref, xs_ref):
    deg = degt_ref[0, :, 0:1] + degt_ref[1, :, 0:1] + 1.0
    dinv = lax.rsqrt(deg)
    dinv_ref[...] = dinv
    xs_ref[...] = x_ref[...] * dinv


def _gates_body(acc_ref, xs_ref, dinv_ref, h_ref,
                Wz_ref, Wr_ref, Wh_ref, Wlz_ref, Wlr_ref, Wlh_ref,
                bz_ref, br_ref, bh_ref, blz_ref, blr_ref, blh_ref,
                Wout_ref, bout_ref, z_ref, H_ref):
    d = xs_ref.shape[1]
    hi = pl.ds(0, d)
    lo = pl.ds(d, d)

    def mm(a, b):
        return lax.dot_general(a, b, (((1,), (0,)), ((), ())),
                               precision=lax.Precision.HIGHEST,
                               preferred_element_type=jnp.float32)

    agg = (acc_ref[0] + acc_ref[1] + xs_ref[...]) * dinv_ref[...]
    h = h_ref[...]

    Az = mm(Wz_ref[...], Wlz_ref[hi, :])
    cz = mm(bz_ref[...], Wlz_ref[hi, :]) + blz_ref[...]
    Z = jax.nn.sigmoid(mm(agg, Az) + mm(h, Wlz_ref[lo, :]) + cz)

    Ar = mm(Wr_ref[...], Wlr_ref[hi, :])
    cr = mm(br_ref[...], Wlr_ref[hi, :]) + blr_ref[...]
    R = jax.nn.sigmoid(mm(agg, Ar) + mm(h, Wlr_ref[lo, :]) + cr)

    Ah = mm(Wh_ref[...], Wlh_ref[hi, :])
    ch = mm(bh_ref[...], Wlh_ref[hi, :]) + blh_ref[...]
    C = jnp.tanh(mm(agg, Ah) + mm(h * R, Wlh_ref[lo, :]) + ch)

    H = Z * h + (1.0 - Z) * C
    H_ref[...] = H
    z_ref[...] = mm(jnp.maximum(H, 0.0), Wout_ref[...]) + bout_ref[...]


def kernel(node_feat, edge_index, edge_weight, h,
           Wz, bz, Wr, br, Wh, bh,
           Wlz, blz, Wlr, blr, Wlh, blh, Wout, bout):
    B, Nn, Din = node_feat.shape
    n = B * Nn
    d = h.shape[1]
    x = node_feat.reshape(n, Din)
    src = edge_index[0]
    dst = edge_index[1]
    E = src.shape[0]

    group = NW * CHUNK * PHR
    epad = ((E + group - 1) // group) * group
    pad = epad - E
    srcp = jnp.concatenate([src, jnp.zeros((pad,), jnp.int32)]).reshape(-1, CHUNK)
    dstp = jnp.concatenate([dst, jnp.zeros((pad,), jnp.int32)]).reshape(-1, CHUNK)
    wp = jnp.concatenate(
        [edge_weight, jnp.zeros((pad,), jnp.float32)]).reshape(-1, CHUNK)
    rows_total = epad // CHUNK

    zeros_nl = jnp.zeros((n, LANE), jnp.float32)
    zeros_nd = jnp.zeros((n, d), jnp.float32)

    degp = _make_deg_kernel(n, rows_total)(dstp, wp, zeros_nl)
    degt = degp.reshape(NC, n, LANE)

    R = 2000
    grid = (n // R,)
    dinv, xs = pl.pallas_call(
        _prep_body,
        grid=grid,
        in_specs=[
            pl.BlockSpec((NC, R, LANE), lambda i: (0, i, 0)),
            pl.BlockSpec((R, Din), lambda i: (i, 0)),
        ],
        out_specs=[
            pl.BlockSpec((R, 1), lambda i: (i, 0)),
            pl.BlockSpec((R, Din), lambda i: (i, 0)),
        ],
        out_shape=[
            jax.ShapeDtypeStruct((n, 1), jnp.float32),
            jax.ShapeDtypeStruct((n, Din), jnp.float32),
        ],
    )(degt, x)

    accp = _make_agg_kernel(n, d, rows_total)(srcp, dstp, wp, xs, zeros_nd)
    accp = accp.reshape(NC, n, d)

    full = lambda *shape: pl.BlockSpec(shape, lambda i: tuple(0 for _ in shape))
    row_blk = lambda c: pl.BlockSpec((R, c), lambda i: (i, 0))
    z, H = pl.pallas_call(
        _gates_body,
        grid=grid,
        in_specs=[
            pl.BlockSpec((NC, R, d), lambda i: (0, i, 0)),
            row_blk(d), row_blk(1), row_blk(d),
            full(d, d), full(d, d), full(d, d),
            full(2 * d, d), full(2 * d, d), full(2 * d, d),
            full(1, d), full(1, d), full(1, d),
            full(1, d), full(1, d), full(1, d),
            full(d, 1), full(1, 1),
        ],
        out_specs=[row_blk(1), row_blk(d)],
        out_shape=[
            jax.ShapeDtypeStruct((n, 1), jnp.float32),
            jax.ShapeDtypeStruct((n, d), jnp.float32),
        ],
    )(accp, xs, dinv, h, Wz, Wr, Wh, Wlz, Wlr, Wlh,
      bz.reshape(1, d), br.reshape(1, d), bh.reshape(1, d),
      blz.reshape(1, d), blr.reshape(1, d), blh.reshape(1, d),
      Wout, bout.reshape(1, 1))

    return z.reshape(B, Nn, 1), H
